# baseline (speedup 1.0000x reference)
import functools

import jax
import jax.numpy as jnp
from jax import lax
from jax.experimental import pallas as pl
from jax.experimental.pallas import tpu as pltpu
from jax.experimental.pallas import tpu_sc as plsc

HIDDEN = 2048
NUM_TOKENS = 4096
NUM_ROWS = 16384

NUM_CORES = 2
NUM_SUBCORES = 16
NUM_WORKERS = NUM_CORES * NUM_SUBCORES
ROWS_PER_WORKER = NUM_ROWS // NUM_WORKERS
C = 16
NCH = ROWS_PER_WORKER // C
NBINS = NUM_ROWS // C
W = 16


def _body(text_hbm, image_hbm, idx_hbm, src_hbm, lo_hbm, out_hbm,
          idx_v, src_v, lov_v, buf_a, buf_b, tsem, wsem, msem):
    wid = lax.axis_index("s") * NUM_CORES + lax.axis_index("c")
    pltpu.sync_copy(idx_hbm, idx_v)
    pltpu.sync_copy(src_hbm, src_v)
    pltpu.sync_copy(lo_hbm.at[wid], lov_v)
    lvecs = [lov_v[pl.ds(0, 16)], lov_v[pl.ds(16, 16)], lov_v[pl.ds(32, 16)]]
    los = [lvecs[j // 16][j % 16] for j in range(NCH + 1)]

    r0 = wid * ROWS_PER_WORKER
    bufs = (buf_a, buf_b)

    def merge_chunk(c, buf):
        chunk_row0 = r0 + c * C
        lo_c, hi_c = los[c], los[c + 1]
        lo8 = lo_c & jnp.int32(~7)
        nwin = lax.select(
            hi_c > lo_c,
            lax.div(hi_c - lo8 + jnp.int32(W - 1), jnp.int32(W)),
            jnp.int32(0),
        )

        def win_body(k, carry):
            base = jnp.minimum(lo8 + k * W, jnp.int32(NUM_TOKENS - W))
            w_idx = idx_v[pl.ds(base, W)]
            w_src = src_v[pl.ds(base, W)]
            loc = jnp.clip(w_idx - chunk_row0, jnp.int32(-1), jnp.int32(C)) + 1
            copies = []
            for m in range(W):
                src_off = w_src[m] * jnp.int32(HIDDEN)
                dst_off = loc[m] * jnp.int32(HIDDEN)
                copies.append(pltpu.async_copy(
                    image_hbm.at[pl.ds(src_off, HIDDEN)],
                    buf.at[pl.ds(dst_off, HIDDEN)],
                    msem))
            for cp in copies:
                cp.wait()
            return carry

        lax.fori_loop(0, nwin, win_body, jnp.int32(0))

    reads = [None] * NCH
    writes = [None] * NCH
    reads[0] = pltpu.async_copy(
        text_hbm.at[pl.ds(r0 * HIDDEN, C * HIDDEN)],
        bufs[0].at[pl.ds(HIDDEN, C * HIDDEN)], tsem)
    for c in range(NCH):
        if c + 1 < NCH:
            if c >= 1:
                writes[c - 1].wait()
            reads[c + 1] = pltpu.async_copy(
                text_hbm.at[pl.ds((r0 + (c + 1) * C) * HIDDEN, C * HIDDEN)],
                bufs[(c + 1) % 2].at[pl.ds(HIDDEN, C * HIDDEN)], tsem)
        reads[c].wait()
        merge_chunk(c, bufs[c % 2])
        writes[c] = pltpu.async_copy(
            bufs[c % 2].at[pl.ds(HIDDEN, C * HIDDEN)],
            out_hbm.at[pl.ds((r0 + c * C) * HIDDEN, C * HIDDEN)], wsem)
    writes[NCH - 1].wait()


@functools.cache
def _get_interleave():
    return pl.kernel(
        _body,
        out_type=jax.ShapeDtypeStruct((NUM_ROWS * HIDDEN,), jnp.float32),
        mesh=plsc.VectorSubcoreMesh(
            core_axis_name="c",
            subcore_axis_name="s",
            num_cores=NUM_CORES,
            num_subcores=NUM_SUBCORES,
        ),
        scratch_types=[
            pltpu.VMEM((NUM_TOKENS,), jnp.int32),
            pltpu.VMEM((NUM_TOKENS,), jnp.int32),
            pltpu.VMEM((48,), jnp.int32),
            pltpu.VMEM(((C + 2) * HIDDEN,), jnp.float32),
            pltpu.VMEM(((C + 2) * HIDDEN,), jnp.float32),
            pltpu.SemaphoreType.DMA,
            pltpu.SemaphoreType.DMA,
            pltpu.SemaphoreType.DMA,
        ],
    )


def kernel(image_embeddings, text_embeddings, vision_indices):
    batch, seq_len, hidden = text_embeddings.shape
    text_1d = jnp.reshape(text_embeddings, (batch * seq_len * hidden,))
    image_1d = jnp.reshape(image_embeddings, (NUM_TOKENS * hidden,))
    idx = vision_indices.astype(jnp.int32)
    iota = jnp.arange(NUM_TOKENS, dtype=jnp.int32)
    nxt = jnp.concatenate([idx[1:], jnp.full((1,), -1, jnp.int32)])
    src = lax.cummin(
        jnp.where(idx != nxt, iota, jnp.int32(NUM_TOKENS)), axis=0, reverse=True
    ).astype(jnp.int32)
    bounds = (jnp.int32(C) * jnp.arange(NBINS + 1, dtype=jnp.int32))[:, None]
    lo_ext = jnp.sum(idx[None, :] < bounds, axis=1, dtype=jnp.int32)
    lo_pad = jnp.concatenate([lo_ext, jnp.zeros((47,), jnp.int32)])
    lo_rows = jnp.stack(
        [lax.slice(lo_pad, (32 * w,), (32 * w + 48,)) for w in range(NUM_WORKERS)]
    )
    out = _get_interleave()(text_1d, image_1d, idx, src, lo_rows)
    return jnp.reshape(out, (batch, seq_len, hidden))
